# Initial kernel scaffold; baseline (speedup 1.0000x reference)
import jax
import jax.numpy as jnp
from jax.experimental import pallas as pl


def kernel(x, edge_index, batch, W1, b1, W2, b2, fc1_W, fc1_b, fc3_W, fc3_b):
    raise NotImplementedError("write your pallas kernel here")


# speedup vs baseline: 8.1813x; 8.1813x over previous
import functools

import jax
import jax.numpy as jnp
from jax import lax
from jax.experimental import pallas as pl
from jax.experimental.pallas import tpu as pltpu
from jax.experimental.pallas import tpu_sc as plsc

N_RAW = 10000
E_RAW = 320000
D = 128
G = 16

NC, NS = 2, 16
NW = NC * NS
N_PAD = 10240
E_PAD = 327680
E_PER_TILE = E_PAD // NW
CHUNK = 128
N_CHUNKS = E_PER_TILE // CHUNK
ROWS_PER_TILE = N_PAD // NS
PAD_NODE = N_RAW + 100

ROW_BLK = 2048
TC_GRID = N_PAD // ROW_BLK

_sc_mesh = plsc.VectorSubcoreMesh(core_axis_name="c", subcore_axis_name="s")


def _zero_vmem(buf, rows, cols):
    def body(i, _):
        def inner(j, _):
            buf[i, pl.ds(j * 16, 16)] = jnp.zeros((16,), jnp.float32)
            return 0
        return lax.fori_loop(0, cols // 16, inner, 0)
    lax.fori_loop(0, rows, body, 0)


def _deg_body(dst_hbm, deg_hbm, dst_v, ones_v, buf_v, acc_sh):
    c = lax.axis_index("c")
    s = lax.axis_index("s")
    wid = s * NC + c
    pltpu.sync_copy(dst_hbm.at[wid], dst_v)
    _zero_vmem(ones_v, CHUNK, D)
    row0 = s * ROWS_PER_TILE
    for k in range(ROWS_PER_TILE // CHUNK):
        pltpu.sync_copy(ones_v, acc_sh.at[pl.ds(row0 + k * CHUNK, CHUNK)])
    def fill(i, _):
        def inner(j, _):
            ones_v[i, pl.ds(j * 16, 16)] = jnp.ones((16,), jnp.float32)
            return 0
        return lax.fori_loop(0, D // 16, inner, 0)
    lax.fori_loop(0, CHUNK, fill, 0)
    plsc.subcore_barrier()
    def chunk(j, _):
        pltpu.sync_copy(ones_v, acc_sh.at[dst_v.at[j]], add=True)
        return 0
    lax.fori_loop(0, N_CHUNKS, chunk, 0)
    plsc.subcore_barrier()
    for k in range(ROWS_PER_TILE // CHUNK):
        r = row0 + k * CHUNK
        pltpu.sync_copy(acc_sh.at[pl.ds(r, CHUNK)], ones_v)

        def compact(i, _):
            buf_v[i, pl.ds(0, 16)] = ones_v[i, pl.ds(0, 16)]
            return 0
        lax.fori_loop(0, CHUNK, compact, 0)
        pltpu.sync_copy(buf_v, deg_hbm.at[c, pl.ds(r, CHUNK)])


_deg_kernel = functools.partial(
    pl.kernel,
    out_type=jax.ShapeDtypeStruct((NC, N_PAD, 16), jnp.float32),
    mesh=_sc_mesh,
    scratch_types=[
        pltpu.VMEM((N_CHUNKS, CHUNK), jnp.int32),
        pltpu.VMEM((CHUNK, D), jnp.float32),
        pltpu.VMEM((CHUNK, 16), jnp.float32),
        pltpu.VMEM_SHARED((N_PAD, D), jnp.float32),
    ],
)(_deg_body)


def _scatter_body(h_hbm, src_hbm, dst_hbm, out_hbm, src_v, dst_v, rows_v, acc_sh, sem):
    c = lax.axis_index("c")
    s = lax.axis_index("s")
    wid = s * NC + c
    pltpu.sync_copy(src_hbm.at[wid], src_v)
    pltpu.sync_copy(dst_hbm.at[wid], dst_v)
    _zero_vmem(rows_v, CHUNK, D)
    row0 = s * ROWS_PER_TILE
    for k in range(ROWS_PER_TILE // CHUNK):
        pltpu.sync_copy(rows_v, acc_sh.at[pl.ds(row0 + k * CHUNK, CHUNK)])
    plsc.subcore_barrier()

    def chunk(j, _):
        pltpu.async_copy(h_hbm.at[src_v.at[j]], rows_v, sem).wait()
        pltpu.sync_copy(rows_v, acc_sh.at[dst_v.at[j]], add=True)
        return 0
    lax.fori_loop(0, N_CHUNKS, chunk, 0)
    plsc.subcore_barrier()
    for k in range(ROWS_PER_TILE // CHUNK):
        r = row0 + k * CHUNK
        pltpu.sync_copy(acc_sh.at[pl.ds(r, CHUNK)], rows_v)
        pltpu.sync_copy(rows_v, out_hbm.at[c, pl.ds(r, CHUNK)])


_scatter_kernel = functools.partial(
    pl.kernel,
    out_type=jax.ShapeDtypeStruct((NC, N_PAD, D), jnp.float32),
    mesh=_sc_mesh,
    scratch_types=[
        pltpu.VMEM((N_CHUNKS, CHUNK), jnp.int32),
        pltpu.VMEM((N_CHUNKS, CHUNK), jnp.int32),
        pltpu.VMEM((CHUNK, D), jnp.float32),
        pltpu.VMEM_SHARED((N_PAD, D), jnp.float32),
        pltpu.SemaphoreType.DMA,
    ],
)(_scatter_body)


def _dinv_from(degp):
    return lax.rsqrt(degp[0, :, 0:1] + degp[1, :, 0:1] + 1.0)


def _tc_h1s_body(degp_ref, x_ref, w1_ref, o_ref):
    dinv = _dinv_from(degp_ref[...])
    h = jnp.dot(x_ref[...], w1_ref[...], preferred_element_type=jnp.float32)
    o_ref[...] = h * dinv


def _tc_mid_body(degp_ref, acc_ref, p1_ref, b1_ref, w2_ref, o_ref):
    dinv = _dinv_from(degp_ref[...])
    a = acc_ref[0] + acc_ref[1] + p1_ref[...]
    h1 = jnp.maximum(dinv * a + b1_ref[...], 0.0)
    o_ref[...] = jnp.dot(h1, w2_ref[...], preferred_element_type=jnp.float32) * dinv


def _tc_final_body(degp_ref, acc_ref, p2_ref, b2_ref, batch_ref,
                   fc1w_ref, fc1b_ref, fc3w_ref, fc3b_ref, o_ref,
                   sums_ref, cnt_ref):
    i = pl.program_id(0)

    @pl.when(i == 0)
    def _():
        sums_ref[...] = jnp.zeros((G, D), jnp.float32)
        cnt_ref[...] = jnp.zeros((G, D), jnp.float32)

    dinv = _dinv_from(degp_ref[...])
    a = acc_ref[0] + acc_ref[1] + p2_ref[...]
    h2 = jnp.maximum(dinv * a + b2_ref[...], 0.0)
    gids = lax.broadcasted_iota(jnp.int32, (1, G), 1)
    mask = (batch_ref[...] == gids).astype(jnp.float32)
    dn = (((0,), (0,)), ((), ()))
    sums_ref[...] += lax.dot_general(mask, h2, dn, preferred_element_type=jnp.float32)
    cnt_ref[...] += lax.dot_general(
        mask, jnp.ones((ROW_BLK, D), jnp.float32), dn,
        preferred_element_type=jnp.float32)

    @pl.when(i == TC_GRID - 1)
    def _():
        hg = sums_ref[...] / jnp.maximum(cnt_ref[...], 1.0)
        t = jnp.maximum(
            jnp.dot(hg, fc1w_ref[...], preferred_element_type=jnp.float32)
            + fc1b_ref[...], 0.0)
        o_ref[...] = (jnp.dot(t, fc3w_ref[...], preferred_element_type=jnp.float32)
                      + fc3b_ref[...])


def _row_spec(cols):
    return pl.BlockSpec((ROW_BLK, cols), lambda i: (i, 0))


_degp_spec = pl.BlockSpec((NC, ROW_BLK, 16), lambda i: (0, i, 0))
_acc_spec = pl.BlockSpec((NC, ROW_BLK, D), lambda i: (0, i, 0))
_full_spec = pl.BlockSpec((D, D), lambda i: (0, 0))
_bias_spec = pl.BlockSpec((1, D), lambda i: (0, 0))


def _tc_h1s(degp, x, w1):
    return pl.pallas_call(
        _tc_h1s_body,
        grid=(TC_GRID,),
        in_specs=[_degp_spec, _row_spec(D), _full_spec],
        out_specs=_row_spec(D),
        out_shape=jax.ShapeDtypeStruct((N_PAD, D), jnp.float32),
    )(degp, x, w1)


def _tc_mid(degp, acc, p1, b1, w2):
    return pl.pallas_call(
        _tc_mid_body,
        grid=(TC_GRID,),
        in_specs=[_degp_spec, _acc_spec, _row_spec(D), _bias_spec, _full_spec],
        out_specs=_row_spec(D),
        out_shape=jax.ShapeDtypeStruct((N_PAD, D), jnp.float32),
    )(degp, acc, p1, b1, w2)


def _tc_final(degp, acc, p2, b2, batch2d, fc1w, fc1b, fc3w, fc3b):
    return pl.pallas_call(
        _tc_final_body,
        grid=(TC_GRID,),
        in_specs=[_degp_spec, _acc_spec, _row_spec(D), _bias_spec,
                  pl.BlockSpec((ROW_BLK, 1), lambda i: (i, 0)),
                  _full_spec, _bias_spec, _full_spec, _bias_spec],
        out_specs=pl.BlockSpec((G, D), lambda i: (0, 0)),
        out_shape=jax.ShapeDtypeStruct((G, D), jnp.float32),
        scratch_shapes=[pltpu.VMEM((G, D), jnp.float32),
                        pltpu.VMEM((G, D), jnp.float32)],
        compiler_params=pltpu.CompilerParams(
            dimension_semantics=("arbitrary",)),
    )(degp, acc, p2, b2, batch2d, fc1w, fc1b, fc3w, fc3b)


def kernel(x, edge_index, batch, W1, b1, W2, b2, fc1_W, fc1_b, fc3_W, fc3_b):
    f32 = jnp.float32
    xp = jnp.zeros((N_PAD, D), f32).at[:N_RAW].set(x)
    batp = jnp.full((N_PAD, 1), -1, jnp.int32).at[:N_RAW, 0].set(batch)
    epad = jnp.full((E_PAD - E_RAW,), PAD_NODE, jnp.int32)
    src = jnp.concatenate([edge_index[0], epad]).reshape(NW, N_CHUNKS, CHUNK)
    dst = jnp.concatenate([edge_index[1], epad]).reshape(NW, N_CHUNKS, CHUNK)

    degp = _deg_kernel(dst)
    p1 = _tc_h1s(degp, xp, W1)
    acc1 = _scatter_kernel(p1, src, dst)
    p2 = _tc_mid(degp, acc1, p1, b1.reshape(1, D), W2)
    acc2 = _scatter_kernel(p2, src, dst)
    return _tc_final(degp, acc2, p2, b2.reshape(1, D), batp,
                     fc1_W, fc1_b.reshape(1, D), fc3_W, fc3_b.reshape(1, D))


# speedup vs baseline: 20.5551x; 2.5124x over previous
import functools

import jax
import jax.numpy as jnp
from jax import lax
from jax.experimental import pallas as pl
from jax.experimental.pallas import tpu as pltpu
from jax.experimental.pallas import tpu_sc as plsc

N_RAW = 10000
E_RAW = 320000
D = 128
G = 16

NC, NS = 2, 16
NW = NC * NS
N_PAD = 10240
E_PAD = 327680
E_PER_TILE = E_PAD // NW
CHUNK = 128
N_CHUNKS = E_PER_TILE // CHUNK
ROWS_PER_TILE = N_PAD // NS
PAD_NODE = N_RAW + 100

ROW_BLK = 2048
TC_GRID = N_PAD // ROW_BLK

_sc_mesh = plsc.VectorSubcoreMesh(core_axis_name="c", subcore_axis_name="s")


def _zero_vmem(buf, rows, cols):
    def body(i, _):
        def inner(j, _):
            buf[i, pl.ds(j * 16, 16)] = jnp.zeros((16,), jnp.float32)
            return 0
        return lax.fori_loop(0, cols // 16, inner, 0)
    lax.fori_loop(0, rows, body, 0)


def _deg_body(dst_hbm, deg_hbm, dst_v, ones_v, buf_v, acc_sh):
    c = lax.axis_index("c")
    s = lax.axis_index("s")
    wid = s * NC + c
    pltpu.sync_copy(dst_hbm.at[wid], dst_v)
    _zero_vmem(ones_v, CHUNK, D)
    row0 = s * ROWS_PER_TILE
    for k in range(ROWS_PER_TILE // CHUNK):
        pltpu.sync_copy(ones_v, acc_sh.at[pl.ds(row0 + k * CHUNK, CHUNK)])
    def fill(i, _):
        def inner(j, _):
            ones_v[i, pl.ds(j * 16, 16)] = jnp.ones((16,), jnp.float32)
            return 0
        return lax.fori_loop(0, D // 16, inner, 0)
    lax.fori_loop(0, CHUNK, fill, 0)
    plsc.subcore_barrier()
    def chunk(j, _):
        pltpu.sync_copy(ones_v, acc_sh.at[dst_v.at[j]], add=True)
        return 0
    lax.fori_loop(0, N_CHUNKS, chunk, 0)
    plsc.subcore_barrier()
    for k in range(ROWS_PER_TILE // CHUNK):
        r = row0 + k * CHUNK
        pltpu.sync_copy(acc_sh.at[pl.ds(r, CHUNK)], ones_v)

        def compact(i, _):
            buf_v[i, pl.ds(0, 16)] = ones_v[i, pl.ds(0, 16)]
            return 0
        lax.fori_loop(0, CHUNK, compact, 0)
        pltpu.sync_copy(buf_v, deg_hbm.at[c, pl.ds(r, CHUNK)])


_deg_kernel = functools.partial(
    pl.kernel,
    out_type=jax.ShapeDtypeStruct((NC, N_PAD, 16), jnp.float32),
    mesh=_sc_mesh,
    scratch_types=[
        pltpu.VMEM((N_CHUNKS, CHUNK), jnp.int32),
        pltpu.VMEM((CHUNK, D), jnp.float32),
        pltpu.VMEM((CHUNK, 16), jnp.float32),
        pltpu.VMEM_SHARED((N_PAD, D), jnp.float32),
    ],
)(_deg_body)


def _scatter_body(h_hbm, src_hbm, dst_hbm, out_hbm, src_v, dst_v, rows_v, acc_sh, sem):
    c = lax.axis_index("c")
    s = lax.axis_index("s")
    wid = s * NC + c
    pltpu.sync_copy(src_hbm.at[wid], src_v)
    pltpu.sync_copy(dst_hbm.at[wid], dst_v)
    _zero_vmem(rows_v, CHUNK, D)
    row0 = s * ROWS_PER_TILE
    for k in range(ROWS_PER_TILE // CHUNK):
        pltpu.sync_copy(rows_v, acc_sh.at[pl.ds(row0 + k * CHUNK, CHUNK)])
    plsc.subcore_barrier()

    def chunk(j, _):
        pltpu.async_copy(h_hbm.at[src_v.at[j]], rows_v, sem).wait()
        pltpu.sync_copy(rows_v, acc_sh.at[dst_v.at[j]], add=True)
        return 0
    lax.fori_loop(0, N_CHUNKS, chunk, 0)
    plsc.subcore_barrier()
    for k in range(ROWS_PER_TILE // CHUNK):
        r = row0 + k * CHUNK
        pltpu.sync_copy(acc_sh.at[pl.ds(r, CHUNK)], rows_v)
        pltpu.sync_copy(rows_v, out_hbm.at[c, pl.ds(r, CHUNK)])


_scatter_kernel = functools.partial(
    pl.kernel,
    out_type=jax.ShapeDtypeStruct((NC, N_PAD, D), jnp.float32),
    mesh=_sc_mesh,
    scratch_types=[
        pltpu.VMEM((N_CHUNKS, CHUNK), jnp.int32),
        pltpu.VMEM((N_CHUNKS, CHUNK), jnp.int32),
        pltpu.VMEM((CHUNK, D), jnp.float32),
        pltpu.VMEM_SHARED((N_PAD, D), jnp.float32),
        pltpu.SemaphoreType.DMA,
    ],
)(_scatter_body)


def _dinv_from(degp):
    return lax.rsqrt(degp[0, :, 0:1] + degp[1, :, 0:1] + 1.0)


def _tc_h1s_body(degp_ref, x_ref, w1_ref, o_ref):
    dinv = _dinv_from(degp_ref[...])
    h = jnp.dot(x_ref[...], w1_ref[...], preferred_element_type=jnp.float32)
    o_ref[...] = h * dinv


def _tc_mid_body(degp_ref, acc_ref, p1_ref, b1_ref, w2_ref, o_ref):
    dinv = _dinv_from(degp_ref[...])
    a = acc_ref[0] + acc_ref[1] + p1_ref[...]
    h1 = jnp.maximum(dinv * a + b1_ref[...], 0.0)
    o_ref[...] = jnp.dot(h1, w2_ref[...], preferred_element_type=jnp.float32) * dinv


def _tc_final_body(degp_ref, acc_ref, p2_ref, b2_ref, batch_ref,
                   fc1w_ref, fc1b_ref, fc3w_ref, fc3b_ref, o_ref,
                   sums_ref, cnt_ref):
    i = pl.program_id(0)

    @pl.when(i == 0)
    def _():
        sums_ref[...] = jnp.zeros((G, D), jnp.float32)
        cnt_ref[...] = jnp.zeros((G, D), jnp.float32)

    dinv = _dinv_from(degp_ref[...])
    a = acc_ref[0] + acc_ref[1] + p2_ref[...]
    h2 = jnp.maximum(dinv * a + b2_ref[...], 0.0)
    gids = lax.broadcasted_iota(jnp.int32, (1, G), 1)
    mask = (batch_ref[...] == gids).astype(jnp.float32)
    dn = (((0,), (0,)), ((), ()))
    sums_ref[...] += lax.dot_general(mask, h2, dn, preferred_element_type=jnp.float32)
    cnt_ref[...] += lax.dot_general(
        mask, jnp.ones((ROW_BLK, D), jnp.float32), dn,
        preferred_element_type=jnp.float32)

    @pl.when(i == TC_GRID - 1)
    def _():
        hg = sums_ref[...] / jnp.maximum(cnt_ref[...], 1.0)
        t = jnp.maximum(
            jnp.dot(hg, fc1w_ref[...], preferred_element_type=jnp.float32)
            + fc1b_ref[...], 0.0)
        o_ref[...] = (jnp.dot(t, fc3w_ref[...], preferred_element_type=jnp.float32)
                      + fc3b_ref[...])


def _row_spec(cols):
    return pl.BlockSpec((ROW_BLK, cols), lambda i: (i, 0))


_degp_spec = pl.BlockSpec((NC, ROW_BLK, 16), lambda i: (0, i, 0))
_acc_spec = pl.BlockSpec((NC, ROW_BLK, D), lambda i: (0, i, 0))
_full_spec = pl.BlockSpec((D, D), lambda i: (0, 0))
_bias_spec = pl.BlockSpec((1, D), lambda i: (0, 0))


def _tc_h1s(degp, x, w1):
    return pl.pallas_call(
        _tc_h1s_body,
        grid=(TC_GRID,),
        in_specs=[_degp_spec, _row_spec(D), _full_spec],
        out_specs=_row_spec(D),
        out_shape=jax.ShapeDtypeStruct((N_PAD, D), jnp.float32),
    )(degp, x, w1)


def _tc_mid(degp, acc, p1, b1, w2):
    return pl.pallas_call(
        _tc_mid_body,
        grid=(TC_GRID,),
        in_specs=[_degp_spec, _acc_spec, _row_spec(D), _bias_spec, _full_spec],
        out_specs=_row_spec(D),
        out_shape=jax.ShapeDtypeStruct((N_PAD, D), jnp.float32),
    )(degp, acc, p1, b1, w2)


def _tc_final(degp, acc, p2, b2, batch2d, fc1w, fc1b, fc3w, fc3b):
    return pl.pallas_call(
        _tc_final_body,
        grid=(TC_GRID,),
        in_specs=[_degp_spec, _acc_spec, _row_spec(D), _bias_spec,
                  pl.BlockSpec((ROW_BLK, 1), lambda i: (i, 0)),
                  _full_spec, _bias_spec, _full_spec, _bias_spec],
        out_specs=pl.BlockSpec((G, D), lambda i: (0, 0)),
        out_shape=jax.ShapeDtypeStruct((G, D), jnp.float32),
        scratch_shapes=[pltpu.VMEM((G, D), jnp.float32),
                        pltpu.VMEM((G, D), jnp.float32)],
        compiler_params=pltpu.CompilerParams(
            dimension_semantics=("arbitrary",)),
    )(degp, acc, p2, b2, batch2d, fc1w, fc1b, fc3w, fc3b)


def kernel(x, edge_index, batch, W1, b1, W2, b2, fc1_W, fc1_b, fc3_W, fc3_b):
    f32 = jnp.float32
    xp = jnp.zeros((N_PAD, D), f32).at[:N_RAW].set(x)
    batp = jnp.full((N_PAD, 1), -1, jnp.int32).at[:N_RAW, 0].set(batch)
    epad = N_RAW + jnp.arange(E_PAD - E_RAW, dtype=jnp.int32) % (N_PAD - N_RAW)
    src = jnp.concatenate([edge_index[0], epad]).reshape(NW, N_CHUNKS, CHUNK)
    dst = jnp.concatenate([edge_index[1], epad]).reshape(NW, N_CHUNKS, CHUNK)

    degp = _deg_kernel(dst)
    p1 = _tc_h1s(degp, xp, W1)
    acc1 = _scatter_kernel(p1, src, dst)
    p2 = _tc_mid(degp, acc1, p1, b1.reshape(1, D), W2)
    acc2 = _scatter_kernel(p2, src, dst)
    return _tc_final(degp, acc2, p2, b2.reshape(1, D), batp,
                     fc1_W, fc1_b.reshape(1, D), fc3_W, fc3_b.reshape(1, D))


# speedup vs baseline: 27.4640x; 1.3361x over previous
import functools

import jax
import jax.numpy as jnp
from jax import lax
from jax.experimental import pallas as pl
from jax.experimental.pallas import tpu as pltpu
from jax.experimental.pallas import tpu_sc as plsc

N_RAW = 10000
E_RAW = 320000
D = 128
G = 16

NC, NS = 2, 16
NW = NC * NS
N_PAD = 10240
E_PAD = 327680
E_PER_TILE = E_PAD // NW
CHUNK = 128
N_CHUNKS = E_PER_TILE // CHUNK
ROWS_PER_TILE = N_PAD // NS
PAD_NODE = N_RAW + 100

ROW_BLK = 2048
TC_GRID = N_PAD // ROW_BLK

_sc_mesh = plsc.VectorSubcoreMesh(core_axis_name="c", subcore_axis_name="s")


def _zero_vmem(buf, rows, cols):
    def body(i, _):
        def inner(j, _):
            buf[i, pl.ds(j * 16, 16)] = jnp.zeros((16,), jnp.float32)
            return 0
        return lax.fori_loop(0, cols // 16, inner, 0)
    lax.fori_loop(0, rows, body, 0)


def _deg_body(dst_hbm, deg_hbm, dst_v, ones_v, buf_v, acc_sh):
    c = lax.axis_index("c")
    s = lax.axis_index("s")
    wid = s * NC + c
    pltpu.sync_copy(dst_hbm.at[wid], dst_v)
    _zero_vmem(ones_v, CHUNK, D)
    row0 = s * ROWS_PER_TILE
    for k in range(ROWS_PER_TILE // CHUNK):
        pltpu.sync_copy(ones_v, acc_sh.at[pl.ds(row0 + k * CHUNK, CHUNK)])
    def fill(i, _):
        def inner(j, _):
            ones_v[i, pl.ds(j * 16, 16)] = jnp.ones((16,), jnp.float32)
            return 0
        return lax.fori_loop(0, D // 16, inner, 0)
    lax.fori_loop(0, CHUNK, fill, 0)
    plsc.subcore_barrier()
    def chunk(j, _):
        pltpu.sync_copy(ones_v, acc_sh.at[dst_v.at[j]], add=True)
        return 0
    lax.fori_loop(0, N_CHUNKS, chunk, 0)
    plsc.subcore_barrier()
    for k in range(ROWS_PER_TILE // CHUNK):
        r = row0 + k * CHUNK
        pltpu.sync_copy(acc_sh.at[pl.ds(r, CHUNK)], ones_v)

        def compact(i, _):
            buf_v[i, pl.ds(0, 16)] = ones_v[i, pl.ds(0, 16)]
            return 0
        lax.fori_loop(0, CHUNK, compact, 0)
        pltpu.sync_copy(buf_v, deg_hbm.at[c, pl.ds(r, CHUNK)])


_deg_kernel = functools.partial(
    pl.kernel,
    out_type=jax.ShapeDtypeStruct((NC, N_PAD, 16), jnp.float32),
    mesh=_sc_mesh,
    scratch_types=[
        pltpu.VMEM((N_CHUNKS, CHUNK), jnp.int32),
        pltpu.VMEM((CHUNK, D), jnp.float32),
        pltpu.VMEM((CHUNK, 16), jnp.float32),
        pltpu.VMEM_SHARED((N_PAD, D), jnp.float32),
    ],
)(_deg_body)


IB = 16
NBLK = N_CHUNKS // IB


def _scatter_body(h_hbm, src_hbm, dst_hbm, out_hbm, src_v, dst_v, rows_v, acc_sh, sem):
    c = lax.axis_index("c")
    s = lax.axis_index("s")
    wid = s * NC + c
    _zero_vmem(rows_v.at[0], CHUNK, D)
    row0 = s * ROWS_PER_TILE
    for k in range(ROWS_PER_TILE // CHUNK):
        pltpu.sync_copy(rows_v.at[0], acc_sh.at[pl.ds(row0 + k * CHUNK, CHUNK)])
    plsc.subcore_barrier()

    for b in range(NBLK):
        pltpu.sync_copy(src_hbm.at[wid, pl.ds(b * IB, IB)], src_v)
        pltpu.sync_copy(dst_hbm.at[wid, pl.ds(b * IB, IB)], dst_v)
        pltpu.async_copy(h_hbm.at[src_v.at[0]], rows_v.at[0], sem)

        def body(i, _):
            for k in range(2):
                j = 2 * i + k

                @pl.when(j + 1 < IB)
                def _():
                    pltpu.async_copy(
                        h_hbm.at[src_v.at[j + 1]], rows_v.at[(k + 1) % 2], sem)
                pltpu.make_async_copy(
                    h_hbm.at[src_v.at[j]], rows_v.at[k], sem).wait()
                pltpu.sync_copy(rows_v.at[k], acc_sh.at[dst_v.at[j]], add=True)
            return 0
        lax.fori_loop(0, IB // 2, body, 0)
    plsc.subcore_barrier()
    for k in range(ROWS_PER_TILE // CHUNK):
        r = row0 + k * CHUNK
        pltpu.sync_copy(acc_sh.at[pl.ds(r, CHUNK)], rows_v.at[0])
        pltpu.sync_copy(rows_v.at[0], out_hbm.at[c, pl.ds(r, CHUNK)])


_scatter_kernel = functools.partial(
    pl.kernel,
    out_type=jax.ShapeDtypeStruct((NC, N_PAD, D), jnp.float32),
    mesh=_sc_mesh,
    scratch_types=[
        pltpu.VMEM((IB, CHUNK), jnp.int32),
        pltpu.VMEM((IB, CHUNK), jnp.int32),
        pltpu.VMEM((2, CHUNK, D), jnp.float32),
        pltpu.VMEM_SHARED((N_PAD, D), jnp.float32),
        pltpu.SemaphoreType.DMA,
    ],
)(_scatter_body)


def _dinv_from(degp):
    return lax.rsqrt(degp[0, :, 0:1] + degp[1, :, 0:1] + 1.0)


def _tc_h1s_body(degp_ref, x_ref, w1_ref, o_ref):
    dinv = _dinv_from(degp_ref[...])
    h = jnp.dot(x_ref[...], w1_ref[...], preferred_element_type=jnp.float32)
    o_ref[...] = h * dinv


def _tc_mid_body(degp_ref, acc_ref, p1_ref, b1_ref, w2_ref, o_ref):
    dinv = _dinv_from(degp_ref[...])
    a = acc_ref[0] + acc_ref[1] + p1_ref[...]
    h1 = jnp.maximum(dinv * a + b1_ref[...], 0.0)
    o_ref[...] = jnp.dot(h1, w2_ref[...], preferred_element_type=jnp.float32) * dinv


def _tc_final_body(degp_ref, acc_ref, p2_ref, b2_ref, batch_ref,
                   fc1w_ref, fc1b_ref, fc3w_ref, fc3b_ref, o_ref,
                   sums_ref, cnt_ref):
    i = pl.program_id(0)

    @pl.when(i == 0)
    def _():
        sums_ref[...] = jnp.zeros((G, D), jnp.float32)
        cnt_ref[...] = jnp.zeros((G, D), jnp.float32)

    dinv = _dinv_from(degp_ref[...])
    a = acc_ref[0] + acc_ref[1] + p2_ref[...]
    h2 = jnp.maximum(dinv * a + b2_ref[...], 0.0)
    gids = lax.broadcasted_iota(jnp.int32, (1, G), 1)
    mask = (batch_ref[...] == gids).astype(jnp.float32)
    dn = (((0,), (0,)), ((), ()))
    sums_ref[...] += lax.dot_general(mask, h2, dn, preferred_element_type=jnp.float32)
    cnt_ref[...] += lax.dot_general(
        mask, jnp.ones((ROW_BLK, D), jnp.float32), dn,
        preferred_element_type=jnp.float32)

    @pl.when(i == TC_GRID - 1)
    def _():
        hg = sums_ref[...] / jnp.maximum(cnt_ref[...], 1.0)
        t = jnp.maximum(
            jnp.dot(hg, fc1w_ref[...], preferred_element_type=jnp.float32)
            + fc1b_ref[...], 0.0)
        o_ref[...] = (jnp.dot(t, fc3w_ref[...], preferred_element_type=jnp.float32)
                      + fc3b_ref[...])


def _row_spec(cols):
    return pl.BlockSpec((ROW_BLK, cols), lambda i: (i, 0))


_degp_spec = pl.BlockSpec((NC, ROW_BLK, 16), lambda i: (0, i, 0))
_acc_spec = pl.BlockSpec((NC, ROW_BLK, D), lambda i: (0, i, 0))
_full_spec = pl.BlockSpec((D, D), lambda i: (0, 0))
_bias_spec = pl.BlockSpec((1, D), lambda i: (0, 0))


def _tc_h1s(degp, x, w1):
    return pl.pallas_call(
        _tc_h1s_body,
        grid=(TC_GRID,),
        in_specs=[_degp_spec, _row_spec(D), _full_spec],
        out_specs=_row_spec(D),
        out_shape=jax.ShapeDtypeStruct((N_PAD, D), jnp.float32),
    )(degp, x, w1)


def _tc_mid(degp, acc, p1, b1, w2):
    return pl.pallas_call(
        _tc_mid_body,
        grid=(TC_GRID,),
        in_specs=[_degp_spec, _acc_spec, _row_spec(D), _bias_spec, _full_spec],
        out_specs=_row_spec(D),
        out_shape=jax.ShapeDtypeStruct((N_PAD, D), jnp.float32),
    )(degp, acc, p1, b1, w2)


def _tc_final(degp, acc, p2, b2, batch2d, fc1w, fc1b, fc3w, fc3b):
    return pl.pallas_call(
        _tc_final_body,
        grid=(TC_GRID,),
        in_specs=[_degp_spec, _acc_spec, _row_spec(D), _bias_spec,
                  pl.BlockSpec((ROW_BLK, 1), lambda i: (i, 0)),
                  _full_spec, _bias_spec, _full_spec, _bias_spec],
        out_specs=pl.BlockSpec((G, D), lambda i: (0, 0)),
        out_shape=jax.ShapeDtypeStruct((G, D), jnp.float32),
        scratch_shapes=[pltpu.VMEM((G, D), jnp.float32),
                        pltpu.VMEM((G, D), jnp.float32)],
        compiler_params=pltpu.CompilerParams(
            dimension_semantics=("arbitrary",)),
    )(degp, acc, p2, b2, batch2d, fc1w, fc1b, fc3w, fc3b)


def kernel(x, edge_index, batch, W1, b1, W2, b2, fc1_W, fc1_b, fc3_W, fc3_b):
    f32 = jnp.float32
    xp = jnp.zeros((N_PAD, D), f32).at[:N_RAW].set(x)
    batp = jnp.full((N_PAD, 1), -1, jnp.int32).at[:N_RAW, 0].set(batch)
    epad = N_RAW + jnp.arange(E_PAD - E_RAW, dtype=jnp.int32) % (N_PAD - N_RAW)
    src = jnp.concatenate([edge_index[0], epad]).reshape(NW, N_CHUNKS, CHUNK)
    dst = jnp.concatenate([edge_index[1], epad]).reshape(NW, N_CHUNKS, CHUNK)

    degp = _deg_kernel(dst)
    p1 = _tc_h1s(degp, xp, W1)
    acc1 = _scatter_kernel(p1, src, dst)
    p2 = _tc_mid(degp, acc1, p1, b1.reshape(1, D), W2)
    acc2 = _scatter_kernel(p2, src, dst)
    return _tc_final(degp, acc2, p2, b2.reshape(1, D), batp,
                     fc1_W, fc1_b.reshape(1, D), fc3_W, fc3_b.reshape(1, D))
